# Initial kernel scaffold; baseline (speedup 1.0000x reference)
import jax
import jax.numpy as jnp
from jax.experimental import pallas as pl


def kernel(x, pos, col_embed, row_embed):
    raise NotImplementedError("write your pallas kernel here")


# speedup vs baseline: 9.7591x; 9.7591x over previous
import jax
import jax.numpy as jnp
from jax import lax
from jax.experimental import pallas as pl
from jax.experimental.pallas import tpu as pltpu

N = 262144
D = 256
HALF = D // 2
TABLE_PAD = 64
BLOCK = 2048
NUM_BLOCKS = N // BLOCK


def _body(pos0_ref, pos1_ref, colw_ref, roww_ref, x_ref, o_ref):
    idx0 = pos0_ref[0, 0, :]
    idx1 = pos1_ref[0, 0, :]
    iota = lax.broadcasted_iota(jnp.int32, (BLOCK, TABLE_PAD), 1)
    oh0 = (idx0[:, None] == iota).astype(jnp.float32)
    oh1 = (idx1[:, None] == iota).astype(jnp.float32)
    emb0 = jnp.dot(oh0, colw_ref[...], preferred_element_type=jnp.float32)
    emb1 = jnp.dot(oh1, roww_ref[...], preferred_element_type=jnp.float32)
    o_ref[:, :HALF] = x_ref[:, :HALF] + emb0
    o_ref[:, HALF:] = x_ref[:, HALF:] + emb1


def kernel(x, pos, col_embed, row_embed):
    pos32 = pos.astype(jnp.int32)
    pos0 = pos32[:, 0].reshape(NUM_BLOCKS, 1, BLOCK)
    pos1 = pos32[:, 1].reshape(NUM_BLOCKS, 1, BLOCK)
    colw = jnp.zeros((TABLE_PAD, HALF), jnp.float32).at[:50].set(col_embed)
    roww = jnp.zeros((TABLE_PAD, HALF), jnp.float32).at[:50].set(row_embed)

    return pl.pallas_call(
        _body,
        grid=(NUM_BLOCKS,),
        in_specs=[
            pl.BlockSpec((1, 1, BLOCK), lambda i: (i, 0, 0)),
            pl.BlockSpec((1, 1, BLOCK), lambda i: (i, 0, 0)),
            pl.BlockSpec((TABLE_PAD, HALF), lambda i: (0, 0)),
            pl.BlockSpec((TABLE_PAD, HALF), lambda i: (0, 0)),
            pl.BlockSpec((BLOCK, D), lambda i: (i, 0)),
        ],
        out_specs=pl.BlockSpec((BLOCK, D), lambda i: (i, 0)),
        out_shape=jax.ShapeDtypeStruct((N, D), jnp.float32),
        compiler_params=pltpu.CompilerParams(
            dimension_semantics=("arbitrary",),
        ),
    )(pos0, pos1, colw, roww, x)


# speedup vs baseline: 11.7197x; 1.2009x over previous
import jax
import jax.numpy as jnp
from jax import lax
from jax.experimental import pallas as pl
from jax.experimental.pallas import tpu as pltpu

N = 262144
D = 256
HALF = D // 2
TABLE_PAD = 64
BLOCK = 4096
NUM_BLOCKS = N // BLOCK


def _body(pos0_ref, pos1_ref, colw_ref, roww_ref, x_ref, o_ref):
    idx0 = pos0_ref[0, 0, :]
    idx1 = pos1_ref[0, 0, :]
    iota = lax.broadcasted_iota(jnp.int32, (BLOCK, TABLE_PAD), 1)
    oh0 = (idx0[:, None] == iota).astype(jnp.float32)
    oh1 = (idx1[:, None] == iota).astype(jnp.float32)
    emb0 = jnp.dot(oh0, colw_ref[...], preferred_element_type=jnp.float32)
    emb1 = jnp.dot(oh1, roww_ref[...], preferred_element_type=jnp.float32)
    o_ref[:, :HALF] = x_ref[:, :HALF] + emb0
    o_ref[:, HALF:] = x_ref[:, HALF:] + emb1


def kernel(x, pos, col_embed, row_embed):
    pos32 = pos.astype(jnp.int32)
    pos0 = pos32[:, 0].reshape(NUM_BLOCKS, 1, BLOCK)
    pos1 = pos32[:, 1].reshape(NUM_BLOCKS, 1, BLOCK)
    colw = jnp.zeros((TABLE_PAD, HALF), jnp.float32).at[:50].set(col_embed)
    roww = jnp.zeros((TABLE_PAD, HALF), jnp.float32).at[:50].set(row_embed)

    return pl.pallas_call(
        _body,
        grid=(NUM_BLOCKS,),
        in_specs=[
            pl.BlockSpec((1, 1, BLOCK), lambda i: (i, 0, 0)),
            pl.BlockSpec((1, 1, BLOCK), lambda i: (i, 0, 0)),
            pl.BlockSpec((TABLE_PAD, HALF), lambda i: (0, 0)),
            pl.BlockSpec((TABLE_PAD, HALF), lambda i: (0, 0)),
            pl.BlockSpec((BLOCK, D), lambda i: (i, 0)),
        ],
        out_specs=pl.BlockSpec((BLOCK, D), lambda i: (i, 0)),
        out_shape=jax.ShapeDtypeStruct((N, D), jnp.float32),
        compiler_params=pltpu.CompilerParams(
            dimension_semantics=("arbitrary",),
        ),
    )(pos0, pos1, colw, roww, x)


# speedup vs baseline: 12.0741x; 1.0302x over previous
import jax
import jax.numpy as jnp
from jax import lax
from jax.experimental import pallas as pl
from jax.experimental.pallas import tpu as pltpu

N = 262144
D = 256
HALF = D // 2
TABLE_PAD = 64
BLOCK = 8192
NUM_BLOCKS = N // BLOCK


def _body(pos0_ref, pos1_ref, colw_ref, roww_ref, x_ref, o_ref):
    idx0 = pos0_ref[0, 0, :]
    idx1 = pos1_ref[0, 0, :]
    iota = lax.broadcasted_iota(jnp.int32, (BLOCK, TABLE_PAD), 1)
    oh0 = (idx0[:, None] == iota).astype(jnp.float32)
    oh1 = (idx1[:, None] == iota).astype(jnp.float32)
    emb0 = jnp.dot(oh0, colw_ref[...], preferred_element_type=jnp.float32)
    emb1 = jnp.dot(oh1, roww_ref[...], preferred_element_type=jnp.float32)
    o_ref[:, :HALF] = x_ref[:, :HALF] + emb0
    o_ref[:, HALF:] = x_ref[:, HALF:] + emb1


def kernel(x, pos, col_embed, row_embed):
    pos32 = pos.astype(jnp.int32)
    pos0 = pos32[:, 0].reshape(NUM_BLOCKS, 1, BLOCK)
    pos1 = pos32[:, 1].reshape(NUM_BLOCKS, 1, BLOCK)
    colw = jnp.zeros((TABLE_PAD, HALF), jnp.float32).at[:50].set(col_embed)
    roww = jnp.zeros((TABLE_PAD, HALF), jnp.float32).at[:50].set(row_embed)

    return pl.pallas_call(
        _body,
        grid=(NUM_BLOCKS,),
        in_specs=[
            pl.BlockSpec((1, 1, BLOCK), lambda i: (i, 0, 0)),
            pl.BlockSpec((1, 1, BLOCK), lambda i: (i, 0, 0)),
            pl.BlockSpec((TABLE_PAD, HALF), lambda i: (0, 0)),
            pl.BlockSpec((TABLE_PAD, HALF), lambda i: (0, 0)),
            pl.BlockSpec((BLOCK, D), lambda i: (i, 0)),
        ],
        out_specs=pl.BlockSpec((BLOCK, D), lambda i: (i, 0)),
        out_shape=jax.ShapeDtypeStruct((N, D), jnp.float32),
        compiler_params=pltpu.CompilerParams(
            dimension_semantics=("arbitrary",),
        ),
    )(pos0, pos1, colw, roww, x)
